# baseline (speedup 1.0000x reference)
import jax
import jax.numpy as jnp
from jax.experimental import pallas as pl
from jax.experimental.pallas import tpu as pltpu

N = 10000
D = 128
H = 128

BM = 200
NB = N // BM


def _dgi_kernel(pos_ref, neg_ref, w_ref, b_ref, a_ref, prelu_ref,
                wb_ref, bb_ref, out_ref, x_ref, h_ref, ssum_ref):
    i = pl.program_id(0)

    @pl.when(i == 0)
    def _build_x():
        w_t = w_ref[...].T
        bvec = b_ref[...]
        xp = jnp.dot(pos_ref[...], w_t, preferred_element_type=jnp.float32) + bvec
        xn = jnp.dot(neg_ref[...], w_t, preferred_element_type=jnp.float32) + bvec
        x_ref[...] = jnp.concatenate([xp, xn], axis=1).astype(jnp.bfloat16)
        ssum_ref[...] = jnp.zeros_like(ssum_ref)

    @pl.when(i < NB)
    def _aggregate():
        acc = jnp.dot(
            a_ref[...].astype(jnp.bfloat16),
            x_ref[...],
            preferred_element_type=jnp.float32,
        )
        p = prelu_ref[0, 0]
        h = jnp.where(acc >= 0, acc, p * acc)
        h_ref[pl.ds(i * BM, BM), :] = h.astype(jnp.bfloat16)
        ssum_ref[...] += jnp.sum(h[:, :H], axis=0, keepdims=True)

    @pl.when(i == NB)
    def _score():
        s = jax.nn.sigmoid(ssum_ref[...] * (1.0 / N))
        v = jnp.dot(s, wb_ref[...].T, preferred_element_type=jnp.float32)
        vb = v.astype(jnp.bfloat16)
        bias = bb_ref[0, 0]
        dn = (((1,), (1,)), ((), ()))
        ps = jax.lax.dot_general(vb, h_ref[:, :H], dn,
                                 preferred_element_type=jnp.float32)
        ns = jax.lax.dot_general(vb, h_ref[:, H:], dn,
                                 preferred_element_type=jnp.float32)
        out_ref[0, :] = ps[0] + bias
        out_ref[1, :] = ns[0] + bias


def kernel(pos, neg, a, W, b, prelu_w, Wb, bb):
    pos2 = pos[0]
    neg2 = neg[0]
    b2 = b.reshape(1, H)
    prelu2 = jnp.reshape(prelu_w, (1, 1)).astype(jnp.float32)
    wb2 = Wb.reshape(H, H)
    bb2 = bb.reshape(1, 1)

    scores = pl.pallas_call(
        _dgi_kernel,
        grid=(NB + 1,),
        in_specs=[
            pl.BlockSpec((N, D), lambda i: (0, 0)),
            pl.BlockSpec((N, D), lambda i: (0, 0)),
            pl.BlockSpec((H, D), lambda i: (0, 0)),
            pl.BlockSpec((1, H), lambda i: (0, 0)),
            pl.BlockSpec((BM, N), lambda i: (jnp.minimum(i, NB - 1), 0)),
            pl.BlockSpec((1, 1), lambda i: (0, 0)),
            pl.BlockSpec((H, H), lambda i: (0, 0)),
            pl.BlockSpec((1, 1), lambda i: (0, 0)),
        ],
        out_specs=pl.BlockSpec((2, N), lambda i: (0, 0)),
        out_shape=jax.ShapeDtypeStruct((2, N), jnp.float32),
        scratch_shapes=[
            pltpu.VMEM((N, 2 * H), jnp.bfloat16),
            pltpu.VMEM((N, 2 * H), jnp.bfloat16),
            pltpu.VMEM((1, H), jnp.float32),
        ],
        compiler_params=pltpu.CompilerParams(
            dimension_semantics=("arbitrary",),
        ),
    )(pos2, neg2, W, b2, a, prelu2, wb2, bb2)

    return scores.reshape(1, 2 * N)


# speedup vs baseline: 1.0165x; 1.0165x over previous
import jax
import jax.numpy as jnp
from jax.experimental import pallas as pl
from jax.experimental.pallas import tpu as pltpu

N = 10000
D = 128
H = 128

BM = 400
NB = N // BM


def _dgi_kernel(pos_ref, neg_ref, w_ref, b_ref, a_ref, prelu_ref,
                wb_ref, bb_ref, out_ref, x_ref, h_ref, ssum_ref):
    i = pl.program_id(0)

    @pl.when(i == 0)
    def _build_x():
        w_t = w_ref[...].T
        bvec = b_ref[...]
        xp = jnp.dot(pos_ref[...], w_t, preferred_element_type=jnp.float32) + bvec
        xn = jnp.dot(neg_ref[...], w_t, preferred_element_type=jnp.float32) + bvec
        x_ref[...] = jnp.concatenate([xp, xn], axis=1).astype(jnp.bfloat16)
        ssum_ref[...] = jnp.zeros_like(ssum_ref)

    @pl.when(i < NB)
    def _aggregate():
        acc = jnp.dot(
            a_ref[...].astype(jnp.bfloat16),
            x_ref[...],
            preferred_element_type=jnp.float32,
        )
        p = prelu_ref[0, 0]
        h = jnp.where(acc >= 0, acc, p * acc)
        h_ref[pl.ds(i * BM, BM), :] = h.astype(jnp.bfloat16)
        ssum_ref[...] += jnp.sum(h[:, :H], axis=0, keepdims=True)

    @pl.when(i == NB)
    def _score():
        s = jax.nn.sigmoid(ssum_ref[...] * (1.0 / N))
        v = jnp.dot(s, wb_ref[...].T, preferred_element_type=jnp.float32)
        vb = v.astype(jnp.bfloat16)
        bias = bb_ref[0, 0]
        dn = (((1,), (1,)), ((), ()))
        ps = jax.lax.dot_general(vb, h_ref[:, :H], dn,
                                 preferred_element_type=jnp.float32)
        ns = jax.lax.dot_general(vb, h_ref[:, H:], dn,
                                 preferred_element_type=jnp.float32)
        out_ref[0, :] = ps[0] + bias
        out_ref[1, :] = ns[0] + bias


def kernel(pos, neg, a, W, b, prelu_w, Wb, bb):
    pos2 = pos[0]
    neg2 = neg[0]
    b2 = b.reshape(1, H)
    prelu2 = jnp.reshape(prelu_w, (1, 1)).astype(jnp.float32)
    wb2 = Wb.reshape(H, H)
    bb2 = bb.reshape(1, 1)

    scores = pl.pallas_call(
        _dgi_kernel,
        grid=(NB + 1,),
        in_specs=[
            pl.BlockSpec((N, D), lambda i: (0, 0)),
            pl.BlockSpec((N, D), lambda i: (0, 0)),
            pl.BlockSpec((H, D), lambda i: (0, 0)),
            pl.BlockSpec((1, H), lambda i: (0, 0)),
            pl.BlockSpec((BM, N), lambda i: (jnp.minimum(i, NB - 1), 0)),
            pl.BlockSpec((1, 1), lambda i: (0, 0)),
            pl.BlockSpec((H, H), lambda i: (0, 0)),
            pl.BlockSpec((1, 1), lambda i: (0, 0)),
        ],
        out_specs=pl.BlockSpec((2, N), lambda i: (0, 0)),
        out_shape=jax.ShapeDtypeStruct((2, N), jnp.float32),
        scratch_shapes=[
            pltpu.VMEM((N, 2 * H), jnp.bfloat16),
            pltpu.VMEM((N, 2 * H), jnp.bfloat16),
            pltpu.VMEM((1, H), jnp.float32),
        ],
        compiler_params=pltpu.CompilerParams(
            dimension_semantics=("arbitrary",),
        ),
    )(pos2, neg2, W, b2, a, prelu2, wb2, bb2)

    return scores.reshape(1, 2 * N)
